# Initial kernel scaffold; baseline (speedup 1.0000x reference)
import jax
import jax.numpy as jnp
from jax.experimental import pallas as pl


def kernel(x, edge_index, edge_attr, batch, W1_root, W1_nbr, b1, gamma1, beta1, W2_root, W2_nbr, b2, gamma2, beta2, W3_root, W3_nbr, b3, W_fc, b_fc, W_out, b_out):
    raise NotImplementedError("write your pallas kernel here")


# speedup vs baseline: 3.3324x; 3.3324x over previous
import functools

import jax
import jax.numpy as jnp
from jax import lax
from jax.experimental import pallas as pl
from jax.experimental.pallas import tpu as pltpu
from jax.experimental.pallas import tpu_sc as plsc

N = 10000
D = 128
H = 128
C = 32
G = 128

NC = 2
NS = 16
NW = NC * NS
L = 16

K = 128
NP = 10240
ROWS_PT = NP // NS
ZROWS = 128


def _lane_splat(vec, lane):
    idx = jnp.full((L, 1), lane, jnp.int32)
    dn = lax.GatherDimensionNumbers(
        offset_dims=(), collapsed_slice_dims=(0,), start_index_map=(0,))
    return lax.gather(vec, idx, dn, (1,),
                      mode=lax.GatherScatterMode.PROMISE_IN_BOUNDS)


def _spmm_body(h_hbm, src_hbm, dst_hbm, w_hbm, out_hbm,
               agg_sh, rows_v, src_v, dst_v, w_v, zbuf, sem):
    cid = lax.axis_index("c")
    sid = lax.axis_index("s")
    wid = cid * NS + sid
    e_per_w = src_hbm.shape[0] // NW
    nchunk = e_per_w // K

    def _zrow(r, carry):
        for k in range(H // L):
            zbuf[r, pl.ds(k * L, L)] = jnp.zeros((L,), jnp.float32)
        return carry

    lax.fori_loop(0, ZROWS, _zrow, 0)
    for j in range(ROWS_PT // ZROWS):
        pltpu.sync_copy(zbuf, agg_sh.at[pl.ds(sid * ROWS_PT + j * ZROWS, ZROWS)])
    plsc.subcore_barrier()

    ebase = wid * e_per_w

    def _chunk(c, carry):
        base = ebase + c * K
        pltpu.sync_copy(src_hbm.at[pl.ds(base, K)], src_v)
        pltpu.sync_copy(dst_hbm.at[pl.ds(base, K)], dst_v)
        pltpu.sync_copy(w_hbm.at[pl.ds(base, K)], w_v)
        pltpu.async_copy(h_hbm.at[src_v], rows_v, sem).wait()

        def _group(g, gcarry):
            wvec = w_v[pl.ds(g * L, L)]
            for lane in range(L):
                ws = _lane_splat(wvec, lane)
                e = g * L + lane
                for k in range(H // L):
                    sl = pl.ds(k * L, L)
                    rows_v[e, sl] = rows_v[e, sl] * ws
            return gcarry

        lax.fori_loop(0, K // L, _group, 0)
        pltpu.sync_copy(rows_v, agg_sh.at[dst_v], add=True)
        return carry

    lax.fori_loop(0, nchunk, _chunk, 0)
    plsc.subcore_barrier()

    for j in range(ROWS_PT // ZROWS):
        sl = pl.ds(sid * ROWS_PT + j * ZROWS, ZROWS)
        pltpu.sync_copy(agg_sh.at[sl], out_hbm.at[cid].at[sl])


def _make_spmm(e_pad):
    mesh = plsc.VectorSubcoreMesh(
        core_axis_name="c", subcore_axis_name="s",
        num_cores=NC, num_subcores=NS)
    return pl.kernel(
        _spmm_body,
        out_type=jax.ShapeDtypeStruct((NC, NP, H), jnp.float32),
        mesh=mesh,
        scratch_types=[
            pltpu.VMEM_SHARED((NP, H), jnp.float32),
            pltpu.VMEM((K, H), jnp.float32),
            pltpu.VMEM((K,), jnp.int32),
            pltpu.VMEM((K,), jnp.int32),
            pltpu.VMEM((K,), jnp.float32),
            pltpu.VMEM((ZROWS, H), jnp.float32),
            pltpu.SemaphoreType.DMA,
        ],
    )


def _lin2_body(x_ref, wr_ref, wn_ref, hr_ref, hn_ref):
    x = x_ref[...]
    hr_ref[...] = jnp.dot(x, wr_ref[...], preferred_element_type=jnp.float32)
    hn_ref[...] = jnp.dot(x, wn_ref[...], preferred_element_type=jnp.float32)


_lin2 = pl.pallas_call(
    _lin2_body,
    out_shape=(jax.ShapeDtypeStruct((N, H), jnp.float32),
               jax.ShapeDtypeStruct((N, H), jnp.float32)),
)


def _bn_relu(hr, agg, b, gamma, beta):
    t = hr + agg[0, :N, :] + agg[1, :N, :] + b
    m = jnp.mean(t, axis=0)
    v = jnp.var(t, axis=0)
    h = (t - m) / jnp.sqrt(v + 1e-5) * gamma + beta
    return jnp.maximum(h, 0.0)


def _bnlin_body(hr_ref, agg_ref, b_ref, g_ref, be_ref, wr_ref, wn_ref,
                hr2_ref, hn2_ref):
    h = _bn_relu(hr_ref[...], agg_ref[...], b_ref[...], g_ref[...], be_ref[...])
    hr2_ref[...] = jnp.dot(h, wr_ref[...], preferred_element_type=jnp.float32)
    hn2_ref[...] = jnp.dot(h, wn_ref[...], preferred_element_type=jnp.float32)


_bnlin = pl.pallas_call(
    _bnlin_body,
    out_shape=(jax.ShapeDtypeStruct((N, H), jnp.float32),
               jax.ShapeDtypeStruct((N, H), jnp.float32)),
)


def _final_body(hr_ref, agg_ref, b_ref, g_ref, be_ref, batch_ref,
                wfc_ref, bfc_ref, wout_ref, bout_ref, out_ref):
    h = _bn_relu(hr_ref[...], agg_ref[...], b_ref[...], g_ref[...], be_ref[...])
    iot = lax.broadcasted_iota(jnp.int32, (N, G), 1)
    oh = jnp.where(batch_ref[...] == iot, 1.0, 0.0)
    pooled = lax.dot_general(oh, h, (((0,), (0,)), ((), ())),
                             preferred_element_type=jnp.float32)
    z = jnp.dot(pooled, wfc_ref[...], preferred_element_type=jnp.float32)
    z = z + bfc_ref[...]
    z = jnp.dot(z, wout_ref[...], preferred_element_type=jnp.float32)
    z = z + bout_ref[...]
    mz = jnp.max(z, axis=1, keepdims=True)
    lse = mz + jnp.log(jnp.sum(jnp.exp(z - mz), axis=1, keepdims=True))
    out_ref[...] = z - lse


_final = pl.pallas_call(
    _final_body,
    out_shape=jax.ShapeDtypeStruct((G, C), jnp.float32),
)


def kernel(x, edge_index, edge_attr, batch,
           W1_root, W1_nbr, b1, gamma1, beta1,
           W2_root, W2_nbr, b2, gamma2, beta2,
           W3_root, W3_nbr, b3,
           W_fc, b_fc, W_out, b_out):
    edge_index = edge_index.reshape(2, -1).astype(jnp.int32)
    e = edge_index.shape[1]
    e_pad = ((e + NW * K - 1) // (NW * K)) * (NW * K)
    pad = e_pad - e
    src = jnp.concatenate([edge_index[0], jnp.zeros((pad,), jnp.int32)])
    dst = jnp.concatenate([edge_index[1], jnp.zeros((pad,), jnp.int32)])
    w = jnp.concatenate([edge_attr.reshape(-1).astype(jnp.float32),
                         jnp.zeros((pad,), jnp.float32)])
    batch_i = batch.astype(jnp.int32).reshape(N, 1)

    spmm = _make_spmm(e_pad)

    hr, hn = _lin2(x, W1_root, W1_nbr)
    agg = spmm(hn, src, dst, w)
    hr, hn = _bnlin(hr, agg, b1, gamma1, beta1, W2_root, W2_nbr)
    agg = spmm(hn, src, dst, w)
    hr, hn = _bnlin(hr, agg, b2, gamma2, beta2, W3_root, W3_nbr)
    agg = spmm(hn, src, dst, w)
    return _final(hr, agg, b3, gamma2, beta2, batch_i,
                  W_fc, b_fc, W_out, b_out)
